# baseline (speedup 1.0000x reference)
import functools

import jax
import jax.numpy as jnp
from jax import lax
from jax.experimental import pallas as pl
from jax.experimental.pallas import tpu as pltpu

_NUM_CLASSES = 21
_THRESH = 0.5
_VAR0, _VAR1 = 0.1, 0.2
_NEGPOS_RATIO = 3


def _mbl_kernel(annos_ref, priorsT_ref, locT_ref, confT_ref, out_ref, *, bb, p, nobj, c):
    f32 = jnp.float32

    pcx = priorsT_ref[0:1, :]
    pcy = priorsT_ref[1:2, :]
    pw = priorsT_ref[2:3, :]
    ph = priorsT_ref[3:4, :]
    px1 = pcx - pw * 0.5
    py1 = pcy - ph * 0.5
    px2 = pcx + pw * 0.5
    py2 = pcy + ph * 0.5
    parea = (px2 - px1) * (py2 - py1)

    annos = annos_ref[...]

    lane_idx = lax.broadcasted_iota(jnp.int32, (bb, p), 1)

    bto = jnp.full((bb, p), -1.0, f32)
    bti = jnp.zeros((bb, p), jnp.int32)
    bpi = []
    for j in range(nobj):
        tx1 = annos[:, j, 0:1]
        ty1 = annos[:, j, 1:2]
        tx2 = annos[:, j, 2:3]
        ty2 = annos[:, j, 3:4]
        tarea = (tx2 - tx1) * (ty2 - ty1)
        iw = jnp.clip(jnp.minimum(tx2, px2) - jnp.maximum(tx1, px1), 0.0, None)
        ih = jnp.clip(jnp.minimum(ty2, py2) - jnp.maximum(ty1, py1), 0.0, None)
        inter = iw * ih
        ov = inter / (tarea + parea - inter)
        upd = ov > bto
        bto = jnp.where(upd, ov, bto)
        bti = jnp.where(upd, j, bti)
        mx = jnp.max(ov, axis=1, keepdims=True)
        cand = jnp.where(ov == mx, lane_idx, p)
        bpi.append(jnp.min(cand, axis=1, keepdims=True))

    for j in range(nobj):
        hit = lane_idx == bpi[j]
        bto = jnp.where(hit, 2.0, bto)
        bti = jnp.where(hit, j, bti)

    labels_p1 = annos[:, :, 4].astype(jnp.int32) + 1
    conf_t = jnp.zeros((bb, p), jnp.int32)
    mx1 = jnp.zeros((bb, p), f32)
    my1 = jnp.zeros((bb, p), f32)
    mx2 = jnp.zeros((bb, p), f32)
    my2 = jnp.zeros((bb, p), f32)
    for j in range(nobj):
        sel = bti == j
        conf_t = jnp.where(sel, labels_p1[:, j : j + 1], conf_t)
        mx1 = jnp.where(sel, annos[:, j, 0:1], mx1)
        my1 = jnp.where(sel, annos[:, j, 1:2], my1)
        mx2 = jnp.where(sel, annos[:, j, 2:3], mx2)
        my2 = jnp.where(sel, annos[:, j, 3:4], my2)
    pos = bto >= _THRESH
    conf_t = jnp.where(pos, conf_t, 0)
    posf = pos.astype(f32)

    gcx = ((mx1 + mx2) * 0.5 - pcx) / (_VAR0 * pw)
    gcy = ((my1 + my2) * 0.5 - pcy) / (_VAR0 * ph)
    gw = jnp.log((mx2 - mx1) / pw) / _VAR1
    gh = jnp.log((my2 - my1) / ph) / _VAR1
    loss_l = jnp.float32(0.0)
    for ci, g in enumerate((gcx, gcy, gw, gh)):
        d = locT_ref[:, ci, :] - g
        ad = jnp.abs(d)
        sl = jnp.where(ad < 1.0, 0.5 * d * d, ad - 0.5)
        loss_l = loss_l + jnp.sum(sl * posf)

    rmax = confT_ref[:, 0, :]
    for ci in range(1, c):
        rmax = jnp.maximum(rmax, confT_ref[:, ci, :])
    ssum = jnp.zeros((bb, p), f32)
    tgt = jnp.zeros((bb, p), f32)
    for ci in range(c):
        x = confT_ref[:, ci, :]
        ssum = ssum + jnp.exp(x - rmax)
        tgt = jnp.where(conf_t == ci, x, tgt)
    lse = jnp.log(ssum) + rmax
    vc = lse - tgt
    v = jnp.where(pos, 0.0, jnp.maximum(vc, 0.0))

    npos = jnp.sum(pos.astype(jnp.int32), axis=1, keepdims=True)
    k = jnp.minimum(_NEGPOS_RATIO * npos, p - 1)

    vb = lax.bitcast_convert_type(v, jnp.int32)
    lo = jnp.zeros((bb, 1), jnp.int32)
    hi = jnp.full((bb, 1), 0x7F800000, jnp.int32)
    for _ in range(31):
        mid = lo + (hi - lo) // 2
        cnt = jnp.sum((vb >= mid).astype(jnp.int32), axis=1, keepdims=True)
        up = cnt >= k
        lo = jnp.where(up, mid, lo)
        hi = jnp.where(up, hi, mid)
    t = lax.bitcast_convert_type(lo, f32)

    gt = v > t
    sum_gt = jnp.sum(jnp.where(gt, v, 0.0), axis=1, keepdims=True)
    cnt_gt = jnp.sum(gt.astype(f32), axis=1, keepdims=True)
    topk = sum_gt + (k.astype(f32) - cnt_gt) * t

    loss_c = jnp.sum(jnp.where(pos, vc, 0.0)) + jnp.sum(topk)

    li = lax.broadcasted_iota(jnp.int32, (1, 1, 3), 2)
    out_ref[...] = jnp.where(
        li == 0, loss_l, jnp.where(li == 1, loss_c, jnp.sum(npos).astype(f32))
    )


@jax.jit
def kernel(loc_pred, conf_pred, prior_boxes, annos):
    b, p, c = conf_pred.shape
    nobj = annos.shape[1]
    bb = 8
    grid = b // bb

    confT = jnp.transpose(conf_pred, (0, 2, 1))
    locT = jnp.transpose(loc_pred, (0, 2, 1))
    priorsT = prior_boxes.T

    parts = pl.pallas_call(
        functools.partial(_mbl_kernel, bb=bb, p=p, nobj=nobj, c=c),
        grid=(grid,),
        in_specs=[
            pl.BlockSpec((bb, nobj, 5), lambda i: (i, 0, 0)),
            pl.BlockSpec((4, p), lambda i: (0, 0)),
            pl.BlockSpec((bb, 4, p), lambda i: (i, 0, 0)),
            pl.BlockSpec((bb, c, p), lambda i: (i, 0, 0)),
        ],
        out_specs=pl.BlockSpec((1, 1, 3), lambda i: (i, 0, 0)),
        out_shape=jax.ShapeDtypeStruct((grid, 1, 3), jnp.float32),
        compiler_params=pltpu.CompilerParams(
            dimension_semantics=(pltpu.GridDimensionSemantics.PARALLEL,),
        ),
    )(annos, priorsT, locT, confT)

    loss_l = jnp.sum(parts[:, 0, 0])
    loss_c = jnp.sum(parts[:, 0, 1])
    n = jnp.sum(parts[:, 0, 2])
    return (loss_l / n, loss_c / n)


# speedup vs baseline: 1.4553x; 1.4553x over previous
import functools

import jax
import jax.numpy as jnp
from jax import lax
from jax.experimental import pallas as pl
from jax.experimental.pallas import tpu as pltpu

_NUM_CLASSES = 21
_THRESH = 0.5
_VAR0, _VAR1 = 0.1, 0.2
_NEGPOS_RATIO = 3


def _mbl_kernel(annos_ref, priorsT_ref, locT_ref, confT_ref, out_ref, *, bb, p, nobj, c):
    f32 = jnp.float32

    pcx = priorsT_ref[0:1, :]
    pcy = priorsT_ref[1:2, :]
    pw = priorsT_ref[2:3, :]
    ph = priorsT_ref[3:4, :]
    px1 = pcx - pw * 0.5
    py1 = pcy - ph * 0.5
    px2 = pcx + pw * 0.5
    py2 = pcy + ph * 0.5
    parea = (px2 - px1) * (py2 - py1)

    annos = annos_ref[...]

    lane_idx = lax.broadcasted_iota(jnp.int32, (bb, p), 1)

    bto = jnp.full((bb, p), -1.0, f32)
    bti = jnp.zeros((bb, p), jnp.int32)
    bpi = []
    for j in range(nobj):
        tx1 = annos[:, j, 0:1]
        ty1 = annos[:, j, 1:2]
        tx2 = annos[:, j, 2:3]
        ty2 = annos[:, j, 3:4]
        tarea = (tx2 - tx1) * (ty2 - ty1)
        iw = jnp.clip(jnp.minimum(tx2, px2) - jnp.maximum(tx1, px1), 0.0, None)
        ih = jnp.clip(jnp.minimum(ty2, py2) - jnp.maximum(ty1, py1), 0.0, None)
        inter = iw * ih
        ov = inter / (tarea + parea - inter)
        upd = ov > bto
        bto = jnp.where(upd, ov, bto)
        bti = jnp.where(upd, j, bti)
        mx = jnp.max(ov, axis=1, keepdims=True)
        cand = jnp.where(ov == mx, lane_idx, p)
        bpi.append(jnp.min(cand, axis=1, keepdims=True))

    for j in range(nobj):
        hit = lane_idx == bpi[j]
        bto = jnp.where(hit, 2.0, bto)
        bti = jnp.where(hit, j, bti)

    labels_p1 = annos[:, :, 4].astype(jnp.int32) + 1
    conf_t = jnp.zeros((bb, p), jnp.int32)
    mx1 = jnp.zeros((bb, p), f32)
    my1 = jnp.zeros((bb, p), f32)
    mx2 = jnp.zeros((bb, p), f32)
    my2 = jnp.zeros((bb, p), f32)
    for j in range(nobj):
        sel = bti == j
        conf_t = jnp.where(sel, labels_p1[:, j : j + 1], conf_t)
        mx1 = jnp.where(sel, annos[:, j, 0:1], mx1)
        my1 = jnp.where(sel, annos[:, j, 1:2], my1)
        mx2 = jnp.where(sel, annos[:, j, 2:3], mx2)
        my2 = jnp.where(sel, annos[:, j, 3:4], my2)
    pos = bto >= _THRESH
    conf_t = jnp.where(pos, conf_t, 0)
    posf = pos.astype(f32)

    loss_l = jnp.float32(0.0)
    for ci, g in enumerate(
        (
            ((mx1 + mx2) * 0.5 - pcx) / (_VAR0 * pw),
            ((my1 + my2) * 0.5 - pcy) / (_VAR0 * ph),
            jnp.log((mx2 - mx1) / pw) / _VAR1,
            jnp.log((my2 - my1) / ph) / _VAR1,
        )
    ):
        d = locT_ref[ci] - g
        ad = jnp.abs(d)
        sl = jnp.where(ad < 1.0, 0.5 * d * d, ad - 0.5)
        loss_l = loss_l + jnp.sum(sl * posf)

    rmax = confT_ref[0]
    for ci in range(1, c):
        rmax = jnp.maximum(rmax, confT_ref[ci])
    ssum = jnp.zeros((bb, p), f32)
    tgt = jnp.zeros((bb, p), f32)
    for ci in range(c):
        x = confT_ref[ci]
        ssum = ssum + jnp.exp(x - rmax)
        tgt = jnp.where(conf_t == ci, x, tgt)
    lse = jnp.log(ssum) + rmax
    vc = lse - tgt
    v = jnp.where(pos, 0.0, jnp.maximum(vc, 0.0))

    npos = jnp.sum(pos.astype(jnp.int32), axis=1, keepdims=True)
    k = jnp.minimum(_NEGPOS_RATIO * npos, p - 1)

    vb = lax.bitcast_convert_type(v, jnp.int32)
    lo = jnp.zeros((bb, 1), jnp.int32)
    hi = jnp.full((bb, 1), 0x7F800000, jnp.int32)
    for _ in range(31):
        mid = lo + (hi - lo) // 2
        cnt = jnp.sum((vb >= mid).astype(jnp.int32), axis=1, keepdims=True)
        up = cnt >= k
        lo = jnp.where(up, mid, lo)
        hi = jnp.where(up, hi, mid)
    t = lax.bitcast_convert_type(lo, f32)

    gt = v > t
    sum_gt = jnp.sum(jnp.where(gt, v, 0.0), axis=1, keepdims=True)
    cnt_gt = jnp.sum(gt.astype(f32), axis=1, keepdims=True)
    topk = sum_gt + (k.astype(f32) - cnt_gt) * t

    loss_c = jnp.sum(jnp.where(pos, vc, 0.0)) + jnp.sum(topk)

    li = lax.broadcasted_iota(jnp.int32, (1, 1, 3), 2)
    out_ref[...] = jnp.where(
        li == 0, loss_l, jnp.where(li == 1, loss_c, jnp.sum(npos).astype(f32))
    )


@jax.jit
def kernel(loc_pred, conf_pred, prior_boxes, annos):
    b, p, c = conf_pred.shape
    nobj = annos.shape[1]
    bb = 8
    grid = b // bb

    confT = jnp.transpose(conf_pred, (2, 0, 1))
    locT = jnp.transpose(loc_pred, (2, 0, 1))
    priorsT = prior_boxes.T

    parts = pl.pallas_call(
        functools.partial(_mbl_kernel, bb=bb, p=p, nobj=nobj, c=c),
        grid=(grid,),
        in_specs=[
            pl.BlockSpec((bb, nobj, 5), lambda i: (i, 0, 0)),
            pl.BlockSpec((4, p), lambda i: (0, 0)),
            pl.BlockSpec((4, bb, p), lambda i: (0, i, 0)),
            pl.BlockSpec((c, bb, p), lambda i: (0, i, 0)),
        ],
        out_specs=pl.BlockSpec((1, 1, 3), lambda i: (i, 0, 0)),
        out_shape=jax.ShapeDtypeStruct((grid, 1, 3), jnp.float32),
        compiler_params=pltpu.CompilerParams(
            dimension_semantics=(pltpu.GridDimensionSemantics.ARBITRARY,),
        ),
    )(annos, priorsT, locT, confT)

    loss_l = jnp.sum(parts[:, 0, 0])
    loss_c = jnp.sum(parts[:, 0, 1])
    n = jnp.sum(parts[:, 0, 2])
    return (loss_l / n, loss_c / n)
